# baseline (speedup 1.0000x reference)
import functools

import jax
import jax.numpy as jnp
from jax import lax
from jax.experimental import pallas as pl
from jax.experimental.pallas import tpu as pltpu
from jax.experimental.pallas import tpu_sc as plsc

NC, NS, LANES = 2, 16, 16
NW = NC * NS
SUB, LN = 8, 128


def _lse_body(emb_ref, out_ref):
    x = emb_ref[...]
    m = jnp.max(x, axis=1, keepdims=True)
    s = jnp.sum(jnp.exp(x - m), axis=1, keepdims=True)
    out_ref[...] = m + jnp.log(s)


def _row_lse(emb):
    V = emb.shape[0]
    out = pl.pallas_call(
        _lse_body,
        out_shape=jax.ShapeDtypeStruct((V, 1), jnp.float32),
    )(emb)
    return out.reshape(V)


def _transpose_body(V, *refs):
    in_ref, out_ref = refs[0], refs[-1]
    for c in range(SUB):
        v = in_ref[:, c, :].T
        rows = min(LN, V - c * LN)
        out_ref[pl.ds(c * LN, rows)] = v[:rows]


def _padded_transpose_slab(out4, tmp, V, N, slab, TB=1280):
    Nk = out4.shape[0]
    nblk = Nk // TB
    base = slab * nblk
    inputs = [out4]
    in_specs = [pl.BlockSpec((TB, SUB, LN), lambda i: (i, 0, 0))]
    kwargs = {}
    if tmp is not None:
        inputs.append(tmp)
        in_specs.append(pl.BlockSpec(memory_space=pl.ANY))
        kwargs["input_output_aliases"] = {1: 0}
    return pl.pallas_call(
        functools.partial(_transpose_body, V),
        grid=(nblk,),
        in_specs=in_specs,
        out_specs=pl.BlockSpec((V, TB), lambda i: (0, base + i)),
        out_shape=jax.ShapeDtypeStruct((V, N), jnp.float32),
        **kwargs,
    )(*inputs)


def _make_sc_gather(N, V, per_w, C):
    nch = per_w // C
    assert nch % 2 == 0 and nch >= 4
    groups = C // LANES
    pairs = (nch - 2) // 2
    mesh = plsc.VectorSubcoreMesh(
        core_axis_name="c", subcore_axis_name="s",
        num_cores=NC, num_subcores=NS)

    @functools.partial(
        pl.kernel,
        out_type=(
            jax.ShapeDtypeStruct((N, SUB, LN), jnp.float32),
            jax.ShapeDtypeStruct((NW, LANES), jnp.float32),
        ),
        mesh=mesh,
        compiler_params=pltpu.CompilerParams(use_tc_tiling_on_sc=False,
                                              needs_layout_passes=False),
        scratch_types=[
            pltpu.VMEM((per_w,), jnp.int32),
            pltpu.VMEM((per_w,), jnp.int32),
            pltpu.VMEM((V,), jnp.float32),
            pltpu.VMEM((C, SUB, LN), jnp.float32),
            pltpu.VMEM((C, SUB, LN), jnp.float32),
            pltpu.VMEM((LANES,), jnp.float32),
            pltpu.SemaphoreType.DMA,
            pltpu.SemaphoreType.DMA,
            pltpu.SemaphoreType.DMA,
        ],
    )
    def sc(emb_hbm, idx_hbm, tgt_hbm, lse_hbm, out_hbm, part_hbm,
           idx_all, tgt_all, lse_v, rows0, rows1, acc_v, gsem, ssem0, ssem1):
        rows = (rows0, rows1)
        ssem = (ssem0, ssem1)
        wid = lax.axis_index("s") * NC + lax.axis_index("c")
        base = wid * per_w
        pltpu.sync_copy(idx_hbm.at[pl.ds(base, per_w)], idx_all)
        pltpu.sync_copy(tgt_hbm.at[pl.ds(base, per_w)], tgt_all)
        pltpu.sync_copy(lse_hbm, lse_v)

        def g_src(loc):
            return emb_hbm.at[idx_all.at[pl.ds(loc, C)]]

        def scatter_start(loc, rbuf, sem):
            pltpu.async_copy(rbuf, out_hbm.at[pl.ds(base + loc, C)], sem)

        def scatter_wait(loc, rbuf, sem):
            pltpu.make_async_copy(
                rbuf, out_hbm.at[pl.ds(base + loc, C)], sem).wait()

        def compute(loc, rbuf, acc):
            for sub in range(groups):
                o2 = loc + sub * LANES
                i_vec = lax.iota(jnp.int32, LANES) + sub * LANES
                t_vec = tgt_all[pl.ds(o2, LANES)]
                v_vec = idx_all[pl.ds(o2, LANES)]
                val = plsc.load_gather(
                    rbuf, [i_vec, t_vec >> 7, t_vec & (LN - 1)])
                ls = plsc.load_gather(lse_v, [v_vec])
                acc = acc + (ls - val)
            return acc

        pltpu.async_copy(g_src(0), rows0, gsem)
        pltpu.make_async_copy(g_src(0), rows0, gsem).wait()
        scatter_start(0, rows0, ssem0)
        pltpu.async_copy(g_src(C), rows1, gsem)
        acc = compute(0, rows0, jnp.zeros((LANES,), jnp.float32))

        def pair(h, acc):
            for gg in (0, 1):
                s = 2 * h + 1 + gg
                p = 1 - gg
                loc = s * C
                rbuf, obuf = rows[p], rows[1 - p]
                pltpu.make_async_copy(g_src(loc), rbuf, gsem).wait()
                scatter_start(loc, rbuf, ssem[p])
                scatter_wait(loc, obuf, ssem[1 - p])
                pltpu.async_copy(g_src(loc + C), obuf, gsem)
                acc = compute(loc, rbuf, acc)
            return acc

        acc = lax.fori_loop(0, pairs, pair, acc)
        loc = (nch - 1) * C
        pltpu.make_async_copy(g_src(loc), rows1, gsem).wait()
        scatter_start(loc, rows1, ssem1)
        scatter_wait(loc, rows0, ssem0)
        acc = compute(loc, rows1, acc)
        scatter_wait(loc, rows1, ssem1)

        acc_v[...] = acc
        pltpu.sync_copy(acc_v, part_hbm.at[wid])

    return sc


def kernel(idx, targets, emb):
    B, L = idx.shape
    V = emb.shape[0]
    N = B * L
    K = 1
    Nk = N // K
    C = 16
    lse = _row_lse(emb)
    emb_p = jnp.pad(emb, ((0, 0), (0, SUB * LN - V))).reshape(V, SUB, LN)
    idx_f = idx.reshape(N)
    tgt_f = targets.reshape(N)
    sc = _make_sc_gather(Nk, V, Nk // NW, C)
    slabs = [sc(emb_p, idx_f[k * Nk:(k + 1) * Nk],
                tgt_f[k * Nk:(k + 1) * Nk], lse) for k in range(K)]
    tmp = None
    for k, (out4, _) in enumerate(slabs):
        tmp = _padded_transpose_slab(out4, tmp, V, N, k)
    loss = sum(jnp.sum(p) for _, p in slabs) / N
    return (tmp.T, loss)


# speedup vs baseline: 1.0323x; 1.0323x over previous
import functools

import jax
import jax.numpy as jnp
from jax import lax
from jax.experimental import pallas as pl
from jax.experimental.pallas import tpu as pltpu
from jax.experimental.pallas import tpu_sc as plsc

NC, NS, LANES = 2, 16, 16
NW = NC * NS
SUB, LN = 8, 128


def _lse_body(emb_ref, out_ref):
    x = emb_ref[...]
    m = jnp.max(x, axis=1, keepdims=True)
    s = jnp.sum(jnp.exp(x - m), axis=1, keepdims=True)
    out_ref[...] = m + jnp.log(s)


def _row_lse(emb):
    V = emb.shape[0]
    out = pl.pallas_call(
        _lse_body,
        out_shape=jax.ShapeDtypeStruct((V, 1), jnp.float32),
    )(emb)
    return out.reshape(V)


def _transpose_body(V, *refs):
    in_ref, out_ref = refs[0], refs[-1]
    for c in range(SUB):
        v = in_ref[:, c, :].T
        rows = min(LN, V - c * LN)
        out_ref[pl.ds(c * LN, rows)] = v[:rows]


def _padded_transpose_slab(out4, tmp, V, N, slab, TB=3200):
    Nk = out4.shape[0]
    nblk = Nk // TB
    base = slab * nblk
    inputs = [out4]
    in_specs = [pl.BlockSpec((TB, SUB, LN), lambda i: (i, 0, 0))]
    kwargs = {}
    if tmp is not None:
        inputs.append(tmp)
        in_specs.append(pl.BlockSpec(memory_space=pl.ANY))
        kwargs["input_output_aliases"] = {1: 0}
    return pl.pallas_call(
        functools.partial(_transpose_body, V),
        grid=(nblk,),
        in_specs=in_specs,
        out_specs=pl.BlockSpec((V, TB), lambda i: (0, base + i)),
        out_shape=jax.ShapeDtypeStruct((V, N), jnp.float32),
        **kwargs,
    )(*inputs)


def _make_sc_gather(N, V, per_w, C):
    nch = per_w // C
    assert nch % 2 == 0 and nch >= 4
    groups = C // LANES
    pairs = (nch - 2) // 2
    mesh = plsc.VectorSubcoreMesh(
        core_axis_name="c", subcore_axis_name="s",
        num_cores=NC, num_subcores=NS)

    @functools.partial(
        pl.kernel,
        out_type=(
            jax.ShapeDtypeStruct((N, SUB, LN), jnp.float32),
            jax.ShapeDtypeStruct((NW, LANES), jnp.float32),
        ),
        mesh=mesh,
        compiler_params=pltpu.CompilerParams(use_tc_tiling_on_sc=False,
                                              needs_layout_passes=False),
        scratch_types=[
            pltpu.VMEM((per_w,), jnp.int32),
            pltpu.VMEM((per_w,), jnp.int32),
            pltpu.VMEM((V,), jnp.float32),
            pltpu.VMEM((C, SUB, LN), jnp.float32),
            pltpu.VMEM((C, SUB, LN), jnp.float32),
            pltpu.VMEM((LANES,), jnp.float32),
            pltpu.SemaphoreType.DMA,
            pltpu.SemaphoreType.DMA,
            pltpu.SemaphoreType.DMA,
        ],
    )
    def sc(emb_hbm, idx_hbm, tgt_hbm, lse_hbm, out_hbm, part_hbm,
           idx_all, tgt_all, lse_v, rows0, rows1, acc_v, gsem, ssem0, ssem1):
        rows = (rows0, rows1)
        ssem = (ssem0, ssem1)
        wid = lax.axis_index("s") * NC + lax.axis_index("c")
        base = wid * per_w
        pltpu.sync_copy(idx_hbm.at[pl.ds(base, per_w)], idx_all)
        pltpu.sync_copy(tgt_hbm.at[pl.ds(base, per_w)], tgt_all)
        pltpu.sync_copy(lse_hbm, lse_v)

        def g_src(loc):
            return emb_hbm.at[idx_all.at[pl.ds(loc, C)]]

        def scatter_start(loc, rbuf, sem):
            pltpu.async_copy(rbuf, out_hbm.at[pl.ds(base + loc, C)], sem)

        def scatter_wait(loc, rbuf, sem):
            pltpu.make_async_copy(
                rbuf, out_hbm.at[pl.ds(base + loc, C)], sem).wait()

        def compute(loc, rbuf, acc):
            for sub in range(groups):
                o2 = loc + sub * LANES
                i_vec = lax.iota(jnp.int32, LANES) + sub * LANES
                t_vec = tgt_all[pl.ds(o2, LANES)]
                v_vec = idx_all[pl.ds(o2, LANES)]
                val = plsc.load_gather(
                    rbuf, [i_vec, t_vec >> 7, t_vec & (LN - 1)])
                ls = plsc.load_gather(lse_v, [v_vec])
                acc = acc + (ls - val)
            return acc

        pltpu.async_copy(g_src(0), rows0, gsem)
        pltpu.make_async_copy(g_src(0), rows0, gsem).wait()
        scatter_start(0, rows0, ssem0)
        pltpu.async_copy(g_src(C), rows1, gsem)
        acc = compute(0, rows0, jnp.zeros((LANES,), jnp.float32))

        def pair(h, acc):
            for gg in (0, 1):
                s = 2 * h + 1 + gg
                p = 1 - gg
                loc = s * C
                rbuf, obuf = rows[p], rows[1 - p]
                pltpu.make_async_copy(g_src(loc), rbuf, gsem).wait()
                scatter_start(loc, rbuf, ssem[p])
                scatter_wait(loc, obuf, ssem[1 - p])
                pltpu.async_copy(g_src(loc + C), obuf, gsem)
                acc = compute(loc, rbuf, acc)
            return acc

        acc = lax.fori_loop(0, pairs, pair, acc)
        loc = (nch - 1) * C
        pltpu.make_async_copy(g_src(loc), rows1, gsem).wait()
        scatter_start(loc, rows1, ssem1)
        scatter_wait(loc, rows0, ssem0)
        acc = compute(loc, rows1, acc)
        scatter_wait(loc, rows1, ssem1)

        acc_v[...] = acc
        pltpu.sync_copy(acc_v, part_hbm.at[wid])

    return sc


def kernel(idx, targets, emb):
    B, L = idx.shape
    V = emb.shape[0]
    N = B * L
    K = 1
    Nk = N // K
    C = 16
    lse = _row_lse(emb)
    emb_p = jnp.pad(emb, ((0, 0), (0, SUB * LN - V))).reshape(V, SUB, LN)
    idx_f = idx.reshape(N)
    tgt_f = targets.reshape(N)
    sc = _make_sc_gather(Nk, V, Nk // NW, C)
    slabs = [sc(emb_p, idx_f[k * Nk:(k + 1) * Nk],
                tgt_f[k * Nk:(k + 1) * Nk], lse) for k in range(K)]
    tmp = None
    for k, (out4, _) in enumerate(slabs):
        tmp = _padded_transpose_slab(out4, tmp, V, N, k)
    loss = sum(jnp.sum(p) for _, p in slabs) / N
    return (tmp.T, loss)


# speedup vs baseline: 1.1226x; 1.0874x over previous
import functools

import jax
import jax.numpy as jnp
from jax import lax
from jax.experimental import pallas as pl
from jax.experimental.pallas import tpu as pltpu
from jax.experimental.pallas import tpu_sc as plsc

NC, NS, LANES = 2, 16, 16
NW = NC * NS
SUB, LN = 8, 128


def _lse_body(emb_ref, out_ref):
    x = emb_ref[...]
    m = jnp.max(x, axis=1, keepdims=True)
    s = jnp.sum(jnp.exp(x - m), axis=1, keepdims=True)
    out_ref[...] = m + jnp.log(s)


def _row_lse(emb):
    V = emb.shape[0]
    out = pl.pallas_call(
        _lse_body,
        out_shape=jax.ShapeDtypeStruct((V, 1), jnp.float32),
    )(emb)
    return out.reshape(V)


def _transpose_body(V, *refs):
    in_ref, out_ref = refs[0], refs[-1]
    for c in range(SUB):
        v = in_ref[:, c, :].T
        rows = min(LN, V - c * LN)
        out_ref[pl.ds(c * LN, rows)] = v[:rows]


def _padded_transpose_slab(out4, tmp, V, N, slab, TB=3200):
    Nk = out4.shape[0]
    nblk = Nk // TB
    base = slab * nblk
    inputs = [out4]
    in_specs = [pl.BlockSpec((TB, SUB, LN), lambda i: (i, 0, 0))]
    kwargs = {}
    if tmp is not None:
        inputs.append(tmp)
        in_specs.append(pl.BlockSpec(memory_space=pl.ANY))
        kwargs["input_output_aliases"] = {1: 0}
    return pl.pallas_call(
        functools.partial(_transpose_body, V),
        grid=(nblk,),
        in_specs=in_specs,
        out_specs=pl.BlockSpec((V, TB), lambda i: (0, base + i)),
        out_shape=jax.ShapeDtypeStruct((V, N), jnp.float32),
        **kwargs,
    )(*inputs)


def _make_sc_gather(N, V, per_w, C):
    nch = per_w // C
    assert nch % 2 == 0 and nch >= 4
    groups = C // LANES
    pairs = (nch - 2) // 2
    mesh = plsc.VectorSubcoreMesh(
        core_axis_name="c", subcore_axis_name="s",
        num_cores=NC, num_subcores=NS)

    @functools.partial(
        pl.kernel,
        out_type=(
            jax.ShapeDtypeStruct((N, SUB, LN), jnp.float32),
            jax.ShapeDtypeStruct((NW, LANES), jnp.float32),
        ),
        mesh=mesh,
        compiler_params=pltpu.CompilerParams(use_tc_tiling_on_sc=False,
                                              needs_layout_passes=False),
        scratch_types=[
            pltpu.VMEM((per_w,), jnp.int32),
            pltpu.VMEM((per_w,), jnp.int32),
            pltpu.VMEM((V,), jnp.float32),
            pltpu.VMEM((C, SUB, LN), jnp.float32),
            pltpu.VMEM((C, SUB, LN), jnp.float32),
            pltpu.VMEM((LANES,), jnp.float32),
            pltpu.SemaphoreType.DMA,
            pltpu.SemaphoreType.DMA,
            pltpu.SemaphoreType.DMA,
        ],
    )
    def sc(emb_hbm, idx_hbm, tgt_hbm, lse_hbm, out_hbm, part_hbm,
           idx_all, tgt_all, lse_v, rows0, rows1, acc_v, gsem, ssem0, ssem1):
        rows = (rows0, rows1)
        ssem = (ssem0, ssem1)
        wid = lax.axis_index("s") * NC + lax.axis_index("c")
        base = wid * per_w
        pltpu.sync_copy(idx_hbm.at[pl.ds(base, per_w)], idx_all)
        pltpu.sync_copy(tgt_hbm.at[pl.ds(base, per_w)], tgt_all)
        pltpu.sync_copy(lse_hbm, lse_v)

        def g_src(loc):
            return emb_hbm.at[idx_all.at[pl.ds(loc, C)]]

        def scatter_start(loc, rbuf, sem):
            pltpu.async_copy(rbuf, out_hbm.at[pl.ds(base + loc, C)], sem)

        def scatter_wait(loc, rbuf, sem):
            pltpu.make_async_copy(
                rbuf, out_hbm.at[pl.ds(base + loc, C)], sem).wait()

        def compute(loc, rbuf, acc):
            for sub in range(groups):
                o2 = loc + sub * LANES
                i_vec = lax.iota(jnp.int32, LANES) + sub * LANES
                t_vec = tgt_all[pl.ds(o2, LANES)]
                v_vec = idx_all[pl.ds(o2, LANES)]
                val = plsc.load_gather(
                    rbuf, [i_vec, t_vec >> 7, t_vec & (LN - 1)])
                ls = plsc.load_gather(lse_v, [v_vec])
                acc = acc + (ls - val)
            return acc

        pltpu.async_copy(g_src(0), rows0, gsem)
        pltpu.make_async_copy(g_src(0), rows0, gsem).wait()
        scatter_start(0, rows0, ssem0)
        pltpu.async_copy(g_src(C), rows1, gsem)
        acc = compute(0, rows0, jnp.zeros((LANES,), jnp.float32))

        def pair(h, acc):
            for gg in (0, 1):
                s = 2 * h + 1 + gg
                p = 1 - gg
                loc = s * C
                rbuf, obuf = rows[p], rows[1 - p]
                pltpu.make_async_copy(g_src(loc), rbuf, gsem).wait()
                scatter_start(loc, rbuf, ssem[p])
                scatter_wait(loc, obuf, ssem[1 - p])
                pltpu.async_copy(g_src(loc + C), obuf, gsem)
                acc = compute(loc, rbuf, acc)
            return acc

        acc = lax.fori_loop(0, pairs, pair, acc)
        loc = (nch - 1) * C
        pltpu.make_async_copy(g_src(loc), rows1, gsem).wait()
        scatter_start(loc, rows1, ssem1)
        scatter_wait(loc, rows0, ssem0)
        acc = compute(loc, rows1, acc)
        scatter_wait(loc, rows1, ssem1)

        acc_v[...] = acc
        pltpu.sync_copy(acc_v, part_hbm.at[wid])

    return sc


def kernel(idx, targets, emb):
    B, L = idx.shape
    V = emb.shape[0]
    N = B * L
    K = 1
    Nk = N // K
    C = 32
    lse = _row_lse(emb)
    emb_p = jnp.pad(emb, ((0, 0), (0, SUB * LN - V))).reshape(V, SUB, LN)
    idx_f = idx.reshape(N)
    tgt_f = targets.reshape(N)
    sc = _make_sc_gather(Nk, V, Nk // NW, C)
    slabs = [sc(emb_p, idx_f[k * Nk:(k + 1) * Nk],
                tgt_f[k * Nk:(k + 1) * Nk], lse) for k in range(K)]
    tmp = None
    for k, (out4, _) in enumerate(slabs):
        tmp = _padded_transpose_slab(out4, tmp, V, N, k)
    loss = sum(jnp.sum(p) for _, p in slabs) / N
    return (tmp.T, loss)


# speedup vs baseline: 1.1236x; 1.0009x over previous
import functools

import jax
import jax.numpy as jnp
from jax import lax
from jax.experimental import pallas as pl
from jax.experimental.pallas import tpu as pltpu
from jax.experimental.pallas import tpu_sc as plsc

NC, NS, LANES = 2, 16, 16
NW = NC * NS
SUB, LN = 8, 128


def _lse_body(emb_ref, out_ref):
    x = emb_ref[...]
    m = jnp.max(x, axis=1, keepdims=True)
    s = jnp.sum(jnp.exp(x - m), axis=1, keepdims=True)
    out_ref[...] = m + jnp.log(s)


def _row_lse(emb):
    V = emb.shape[0]
    out = pl.pallas_call(
        _lse_body,
        out_shape=jax.ShapeDtypeStruct((V, 1), jnp.float32),
    )(emb)
    return out.reshape(V)


def _transpose_body(V, *refs):
    in_ref, out_ref = refs[0], refs[-1]
    for c in range(SUB):
        v = in_ref[:, c, :].T
        rows = min(LN, V - c * LN)
        out_ref[pl.ds(c * LN, rows)] = v[:rows]


def _padded_transpose_slab(out4, tmp, V, N, slab, TB=3200):
    Nk = out4.shape[0]
    nblk = Nk // TB
    base = slab * nblk
    inputs = [out4]
    in_specs = [pl.BlockSpec((TB, SUB, LN), lambda i: (i, 0, 0))]
    kwargs = {}
    if tmp is not None:
        inputs.append(tmp)
        in_specs.append(pl.BlockSpec(memory_space=pl.ANY))
        kwargs["input_output_aliases"] = {1: 0}
    return pl.pallas_call(
        functools.partial(_transpose_body, V),
        grid=(nblk,),
        in_specs=in_specs,
        out_specs=pl.BlockSpec((V, TB), lambda i: (0, base + i)),
        out_shape=jax.ShapeDtypeStruct((V, N), jnp.float32),
        **kwargs,
    )(*inputs)


def _make_sc_gather(N, V, per_w, C):
    nch = per_w // C
    assert nch * C == per_w and nch >= 4
    groups = C // LANES
    pairs = (nch - 2) // 2
    mesh = plsc.VectorSubcoreMesh(
        core_axis_name="c", subcore_axis_name="s",
        num_cores=NC, num_subcores=NS)

    @functools.partial(
        pl.kernel,
        out_type=(
            jax.ShapeDtypeStruct((N, SUB, LN), jnp.float32),
            jax.ShapeDtypeStruct((NW, LANES), jnp.float32),
        ),
        mesh=mesh,
        compiler_params=pltpu.CompilerParams(use_tc_tiling_on_sc=False,
                                              needs_layout_passes=False),
        scratch_types=[
            pltpu.VMEM((per_w,), jnp.int32),
            pltpu.VMEM((per_w,), jnp.int32),
            pltpu.VMEM((V,), jnp.float32),
            pltpu.VMEM((C, SUB, LN), jnp.float32),
            pltpu.VMEM((C, SUB, LN), jnp.float32),
            pltpu.VMEM((LANES,), jnp.float32),
            pltpu.SemaphoreType.DMA,
            pltpu.SemaphoreType.DMA,
            pltpu.SemaphoreType.DMA,
        ],
    )
    def sc(emb_hbm, idx_hbm, tgt_hbm, lse_hbm, out_hbm, part_hbm,
           idx_all, tgt_all, lse_v, rows0, rows1, acc_v, gsem, ssem0, ssem1):
        rows = (rows0, rows1)
        ssem = (ssem0, ssem1)
        wid = lax.axis_index("s") * NC + lax.axis_index("c")
        base = wid * per_w
        pltpu.sync_copy(idx_hbm.at[pl.ds(base, per_w)], idx_all)
        pltpu.sync_copy(tgt_hbm.at[pl.ds(base, per_w)], tgt_all)
        pltpu.sync_copy(lse_hbm, lse_v)

        def g_src(loc):
            return emb_hbm.at[idx_all.at[pl.ds(loc, C)]]

        def scatter_start(loc, rbuf, sem):
            pltpu.async_copy(rbuf, out_hbm.at[pl.ds(base + loc, C)], sem)

        def scatter_wait(loc, rbuf, sem):
            pltpu.make_async_copy(
                rbuf, out_hbm.at[pl.ds(base + loc, C)], sem).wait()

        def compute(loc, rbuf, acc):
            for sub in range(groups):
                o2 = loc + sub * LANES
                i_vec = lax.iota(jnp.int32, LANES) + sub * LANES
                t_vec = tgt_all[pl.ds(o2, LANES)]
                v_vec = idx_all[pl.ds(o2, LANES)]
                val = plsc.load_gather(
                    rbuf, [i_vec, t_vec >> 7, t_vec & (LN - 1)])
                ls = plsc.load_gather(lse_v, [v_vec])
                acc = acc + (ls - val)
            return acc

        pltpu.async_copy(g_src(0), rows0, gsem)
        pltpu.make_async_copy(g_src(0), rows0, gsem).wait()
        scatter_start(0, rows0, ssem0)
        pltpu.async_copy(g_src(C), rows1, gsem)
        acc = compute(0, rows0, jnp.zeros((LANES,), jnp.float32))

        def pair(h, acc):
            for gg in (0, 1):
                s = 2 * h + 1 + gg
                p = 1 - gg
                loc = s * C
                rbuf, obuf = rows[p], rows[1 - p]
                pltpu.make_async_copy(g_src(loc), rbuf, gsem).wait()
                scatter_start(loc, rbuf, ssem[p])
                scatter_wait(loc, obuf, ssem[1 - p])
                pltpu.async_copy(g_src(loc + C), obuf, gsem)
                acc = compute(loc, rbuf, acc)
            return acc

        acc = lax.fori_loop(0, pairs, pair, acc)
        for s in range(2 * pairs + 1, nch):
            loc = s * C
            b = s % 2
            rbuf, obuf = rows[b], rows[1 - b]
            pltpu.make_async_copy(g_src(loc), rbuf, gsem).wait()
            scatter_start(loc, rbuf, ssem[b])
            scatter_wait(loc, obuf, ssem[1 - b])
            if s + 1 < nch:
                pltpu.async_copy(g_src(loc + C), obuf, gsem)
            acc = compute(loc, rbuf, acc)
        lb = (nch - 1) % 2
        scatter_wait((nch - 1) * C, rows[lb], ssem[lb])

        acc_v[...] = acc
        pltpu.sync_copy(acc_v, part_hbm.at[wid])

    return sc


def kernel(idx, targets, emb):
    B, L = idx.shape
    V = emb.shape[0]
    N = B * L
    K = 1
    Nk = N // K
    C = 32
    lse = _row_lse(emb)
    emb_p = jnp.pad(emb, ((0, 0), (0, SUB * LN - V))).reshape(V, SUB, LN)
    idx_f = idx.reshape(N)
    tgt_f = targets.reshape(N)
    sc = _make_sc_gather(Nk, V, Nk // NW, C)
    slabs = [sc(emb_p, idx_f[k * Nk:(k + 1) * Nk],
                tgt_f[k * Nk:(k + 1) * Nk], lse) for k in range(K)]
    tmp = None
    for k, (out4, _) in enumerate(slabs):
        tmp = _padded_transpose_slab(out4, tmp, V, N, k)
    loss = sum(jnp.sum(p) for _, p in slabs) / N
    return (tmp.T, loss)


# speedup vs baseline: 1.1279x; 1.0038x over previous
import functools

import jax
import jax.numpy as jnp
from jax import lax
from jax.experimental import pallas as pl
from jax.experimental.pallas import tpu as pltpu
from jax.experimental.pallas import tpu_sc as plsc

NC, NS, LANES = 2, 16, 16
NW = NC * NS
SUB, LN = 8, 128


def _lse_body(emb_ref, out_ref):
    x = emb_ref[...]
    m = jnp.max(x, axis=1, keepdims=True)
    s = jnp.sum(jnp.exp(x - m), axis=1, keepdims=True)
    out_ref[...] = m + jnp.log(s)


def _row_lse(emb):
    V = emb.shape[0]
    out = pl.pallas_call(
        _lse_body,
        out_shape=jax.ShapeDtypeStruct((V, 1), jnp.float32),
    )(emb)
    return out.reshape(V)


def _transpose_body(V, *refs):
    in_ref, out_ref = refs[0], refs[-1]
    for c in range(SUB):
        v = in_ref[:, c, :].T
        rows = min(LN, V - c * LN)
        out_ref[pl.ds(c * LN, rows)] = v[:rows]


def _padded_transpose_slab(out4, tmp, V, N, slab, TB=3200):
    Nk = out4.shape[0]
    nblk = Nk // TB
    base = slab * nblk
    inputs = [out4]
    in_specs = [pl.BlockSpec((TB, SUB, LN), lambda i: (i, 0, 0))]
    kwargs = {}
    if tmp is not None:
        inputs.append(tmp)
        in_specs.append(pl.BlockSpec(memory_space=pl.ANY))
        kwargs["input_output_aliases"] = {1: 0}
    return pl.pallas_call(
        functools.partial(_transpose_body, V),
        grid=(nblk,),
        in_specs=in_specs,
        out_specs=pl.BlockSpec((V, TB), lambda i: (0, base + i)),
        out_shape=jax.ShapeDtypeStruct((V, N), jnp.float32),
        **kwargs,
    )(*inputs)


def _make_sc_gather(N, V, per_w, C):
    nch = per_w // C
    assert nch * C == per_w and nch >= 4
    groups = C // LANES
    pairs = (nch - 2) // 2
    mesh = plsc.VectorSubcoreMesh(
        core_axis_name="c", subcore_axis_name="s",
        num_cores=NC, num_subcores=NS)

    @functools.partial(
        pl.kernel,
        out_type=(
            jax.ShapeDtypeStruct((N, SUB, LN), jnp.float32),
            jax.ShapeDtypeStruct((NW, LANES), jnp.float32),
        ),
        mesh=mesh,
        compiler_params=pltpu.CompilerParams(use_tc_tiling_on_sc=False,
                                              needs_layout_passes=False),
        scratch_types=[
            pltpu.VMEM((per_w,), jnp.int32),
            pltpu.VMEM((per_w,), jnp.int32),
            pltpu.VMEM((V,), jnp.float32),
            pltpu.VMEM((C, SUB, LN), jnp.float32),
            pltpu.VMEM((C, SUB, LN), jnp.float32),
            pltpu.VMEM((LANES,), jnp.float32),
            pltpu.SemaphoreType.DMA,
            pltpu.SemaphoreType.DMA,
            pltpu.SemaphoreType.DMA,
        ],
    )
    def sc(emb_hbm, idx_hbm, tgt_hbm, lse_hbm, out_hbm, part_hbm,
           idx_all, tgt_all, lse_v, rows0, rows1, acc_v, gsem, ssem0, ssem1):
        rows = (rows0, rows1)
        ssem = (ssem0, ssem1)
        wid = lax.axis_index("s") * NC + lax.axis_index("c")
        base = wid * per_w
        pltpu.sync_copy(idx_hbm.at[pl.ds(base, per_w)], idx_all)
        pltpu.sync_copy(tgt_hbm.at[pl.ds(base, per_w)], tgt_all)
        pltpu.sync_copy(lse_hbm, lse_v)

        def g_src(loc):
            return emb_hbm.at[idx_all.at[pl.ds(loc, C)]]

        def scatter_start(loc, rbuf, sem):
            pltpu.async_copy(rbuf, out_hbm.at[pl.ds(base + loc, C)], sem)

        def scatter_wait(loc, rbuf, sem):
            pltpu.make_async_copy(
                rbuf, out_hbm.at[pl.ds(base + loc, C)], sem).wait()

        def compute(loc, rbuf, acc):
            for sub in range(groups):
                o2 = loc + sub * LANES
                i_vec = lax.iota(jnp.int32, LANES) + sub * LANES
                t_vec = tgt_all[pl.ds(o2, LANES)]
                v_vec = idx_all[pl.ds(o2, LANES)]
                val = plsc.load_gather(
                    rbuf, [i_vec, t_vec >> 7, t_vec & (LN - 1)])
                ls = plsc.load_gather(lse_v, [v_vec])
                acc = acc + (ls - val)
            return acc

        pltpu.async_copy(g_src(0), rows0, gsem)
        pltpu.make_async_copy(g_src(0), rows0, gsem).wait()
        scatter_start(0, rows0, ssem0)
        pltpu.async_copy(g_src(C), rows1, gsem)
        acc = compute(0, rows0, jnp.zeros((LANES,), jnp.float32))

        def pair(h, acc):
            for gg in (0, 1):
                s = 2 * h + 1 + gg
                p = 1 - gg
                loc = s * C
                rbuf, obuf = rows[p], rows[1 - p]
                pltpu.make_async_copy(g_src(loc), rbuf, gsem).wait()
                scatter_start(loc, rbuf, ssem[p])
                scatter_wait(loc, obuf, ssem[1 - p])
                pltpu.async_copy(g_src(loc + C), obuf, gsem)
                acc = compute(loc, rbuf, acc)
            return acc

        acc = lax.fori_loop(0, pairs, pair, acc)
        for s in range(2 * pairs + 1, nch):
            loc = s * C
            b = s % 2
            rbuf, obuf = rows[b], rows[1 - b]
            pltpu.make_async_copy(g_src(loc), rbuf, gsem).wait()
            scatter_start(loc, rbuf, ssem[b])
            scatter_wait(loc, obuf, ssem[1 - b])
            if s + 1 < nch:
                pltpu.async_copy(g_src(loc + C), obuf, gsem)
            acc = compute(loc, rbuf, acc)
        lb = (nch - 1) % 2
        scatter_wait((nch - 1) * C, rows[lb], ssem[lb])

        acc_v[...] = acc
        pltpu.sync_copy(acc_v, part_hbm.at[wid])

    return sc


def kernel(idx, targets, emb):
    B, L = idx.shape
    V = emb.shape[0]
    N = B * L
    K = 2
    Nk = N // K
    C = 32
    lse = _row_lse(emb)
    emb_p = jnp.pad(emb, ((0, 0), (0, SUB * LN - V))).reshape(V, SUB, LN)
    idx_f = idx.reshape(N)
    tgt_f = targets.reshape(N)
    sc = _make_sc_gather(Nk, V, Nk // NW, C)
    slabs = [sc(emb_p, idx_f[k * Nk:(k + 1) * Nk],
                tgt_f[k * Nk:(k + 1) * Nk], lse) for k in range(K)]
    tmp = None
    for k, (out4, _) in enumerate(slabs):
        tmp = _padded_transpose_slab(out4, tmp, V, N, k)
    loss = sum(jnp.sum(p) for _, p in slabs) / N
    return (tmp.T, loss)
